# Initial kernel scaffold; baseline (speedup 1.0000x reference)
import jax
import jax.numpy as jnp
from jax.experimental import pallas as pl


def kernel(x_phylonodes_up, x_phylonodes_down, x_godnode, edge_index_up_up, edge_index_down_down, edge_index_down_up, edge_index_up_down, edge_index_down_god, edge_index_up_god, params):
    raise NotImplementedError("write your pallas kernel here")


# speedup vs baseline: 7.2476x; 7.2476x over previous
import functools

import jax
import jax.numpy as jnp
from jax import lax
from jax.experimental import pallas as pl
from jax.experimental.pallas import tpu as pltpu
from jax.experimental.pallas import tpu_sc as plsc

N = 10000
NP = 10240
NG = 64
DF = 128
HID = 256
OUT = 128
E = 320000
MAXD = 10
NC = 2
NS = 16
CHUNK = 128
NB_DEG = NP
NB_PAIR = 655360
B = 400
PREC = jax.lax.Precision.DEFAULT
PREC_F32 = jax.lax.Precision.HIGHEST


def _sc_mesh():
    return plsc.VectorSubcoreMesh(
        core_axis_name="c", subcore_axis_name="s", num_cores=NC, num_subcores=NS)


def _build_bincount(nbins, use_pair):
    slab = nbins // NS
    nchunks = E // CHUNK
    ntiles = NC * NS
    iters = (nchunks + ntiles - 1) // ntiles
    assert slab % CHUNK == 0

    def body(src_hbm, dst_hbm, zeros_hbm, out_hbm, src_v, dst_v, bin_v, ones_v, acc, sem):
        del sem
        cid = lax.axis_index("c")
        sid = lax.axis_index("s")
        tid = cid * NS + sid
        pltpu.sync_copy(zeros_hbm, acc.at[pl.ds(sid * slab, slab)])
        for k in range(CHUNK // 16):
            ones_v[pl.ds(k * 16, 16)] = jnp.full((16,), 1.0, jnp.float32)
        plsc.subcore_barrier()

        def chunk_body(i, carry):
            chunk = tid + i * ntiles

            @pl.when(chunk < nchunks)
            def _():
                off = chunk * CHUNK
                pltpu.sync_copy(dst_hbm.at[pl.ds(off, CHUNK)], dst_v)
                if use_pair:
                    pltpu.sync_copy(src_hbm.at[pl.ds(off, CHUNK)], src_v)
                    for k in range(CHUNK // 16):
                        s = src_v[pl.ds(k * 16, 16)]
                        d = dst_v[pl.ds(k * 16, 16)]
                        bin_v[pl.ds(k * 16, 16)] = s * NG + d
                    pltpu.sync_copy(ones_v, acc.at[bin_v], add=True)
                else:
                    pltpu.sync_copy(ones_v, acc.at[dst_v], add=True)

            return carry

        lax.fori_loop(0, iters, chunk_body, 0)
        plsc.subcore_barrier()
        pltpu.sync_copy(acc.at[pl.ds(sid * slab, slab)],
                        out_hbm.at[pl.ds(cid * nbins + sid * slab, slab)])

    return pl.kernel(
        body,
        out_type=jax.ShapeDtypeStruct((NC * nbins,), jnp.float32),
        mesh=_sc_mesh(),
        scratch_types=[
            pltpu.VMEM((CHUNK,), jnp.int32),
            pltpu.VMEM((CHUNK,), jnp.int32),
            pltpu.VMEM((CHUNK,), jnp.int32),
            pltpu.VMEM((CHUNK,), jnp.float32),
            pltpu.VMEM_SHARED((nbins,), jnp.float32),
            pltpu.SemaphoreType.DMA,
        ],
    )


def _build_segsum(feature_split, nrows=N, nedges=E):
    W = 128
    rows = NP // NS
    nchunks = nedges // CHUNK
    ntiles = NS if feature_split else NC * NS
    iters = (nchunks + ntiles - 1) // ntiles

    def body(xa, xb, src_hbm, dst_hbm, zeros_hbm, out_hbm,
             src_v, dst_v, rows_v, acc, sem):
        cid = lax.axis_index("c")
        sid = lax.axis_index("s")
        tix = sid if feature_split else cid * NS + sid
        r0 = sid * rows
        pltpu.sync_copy(zeros_hbm, acc.at[pl.ds(r0, rows), :])
        plsc.subcore_barrier()

        def chunk_body(i, carry):
            chunk = tix + i * ntiles

            @pl.when(chunk < nchunks)
            def _():
                off = chunk * CHUNK
                pltpu.sync_copy(src_hbm.at[pl.ds(off, CHUNK)], src_v)
                pltpu.sync_copy(dst_hbm.at[pl.ds(off, CHUNK)], dst_v)

                @pl.when(cid == 0)
                def _():
                    pltpu.async_copy(xa.at[src_v], rows_v, sem).wait()

                @pl.when(cid == 1)
                def _():
                    pltpu.async_copy(xb.at[src_v], rows_v, sem).wait()

                pltpu.sync_copy(rows_v, acc.at[dst_v], add=True)

            return carry

        lax.fori_loop(0, iters, chunk_body, 0)
        plsc.subcore_barrier()
        pltpu.sync_copy(acc.at[pl.ds(r0, rows), :],
                        out_hbm.at[cid, pl.ds(r0, rows), :])

    return pl.kernel(
        body,
        out_type=jax.ShapeDtypeStruct((NC, NP, W), jnp.float32),
        mesh=_sc_mesh(),
        scratch_types=[
            pltpu.VMEM((CHUNK,), jnp.int32),
            pltpu.VMEM((CHUNK,), jnp.int32),
            pltpu.VMEM((CHUNK, W), jnp.float32),
            pltpu.VMEM_SHARED((NP, W), jnp.float32),
            pltpu.SemaphoreType.DMA,
        ],
    )

def _mf_layer(h1a, h1b, h2a, h2b, x, d1a, d1b, d2a, d2b,
              mf1, mf2, lin, head, interpret=False):
    din = x.shape[1]
    concat = din > 128
    nblk = N // B
    dout = OUT if head is not None else HID

    def body(h1a_r, h1b_r, h2a_r, h2b_r, x_r, d1a_r, d1b_r, d2a_r, d2b_r,
             wl1_r, bl1_r, wr1_r, wl2_r, bl2_r, wr2_r,
             wlin_r, blin_r, *rest):
        if head is not None:
            w2_r, b2_r, o_r = rest
        else:
            (o_r,) = rest
        if concat:
            h1 = jnp.concatenate([h1a_r[...], h1b_r[...]], axis=1)
            h2 = jnp.concatenate([h2a_r[...], h2b_r[...]], axis=1)
        else:
            h1 = h1a_r[...] + h1b_r[...]
            h2 = h2a_r[...] + h2b_r[...]
        xv = x_r[...]
        deg1 = jnp.clip(d1a_r[...] + d1b_r[...], 0.0, float(MAXD)).astype(jnp.int32)
        deg2 = jnp.clip(d2a_r[...] + d2b_r[...], 0.0, float(MAXD)).astype(jnp.int32)
        iota = lax.broadcasted_iota(jnp.int32, (B, MAXD + 1), 1)
        oh1 = (deg1 == iota).astype(jnp.float32)
        oh2 = (deg2 == iota).astype(jnp.float32)
        acc = jnp.zeros((B, HID), jnp.float32)
        for d in range(MAXD + 1):
            t1 = (jnp.dot(h1, wl1_r[d], preferred_element_type=jnp.float32, precision=PREC)
                  + bl1_r[d]
                  + jnp.dot(xv, wr1_r[d], preferred_element_type=jnp.float32, precision=PREC))
            t2 = (jnp.dot(h2, wl2_r[d], preferred_element_type=jnp.float32, precision=PREC)
                  + bl2_r[d]
                  + jnp.dot(xv, wr2_r[d], preferred_element_type=jnp.float32, precision=PREC))
            t1 = t1.astype(jnp.bfloat16).astype(jnp.float32)
            t2 = t2.astype(jnp.bfloat16).astype(jnp.float32)
            acc = acc + oh1[:, d:d + 1] * t1 + oh2[:, d:d + 1] * t2
        y = jnp.dot(acc, wlin_r[...], preferred_element_type=jnp.float32, precision=PREC) + blin_r[...]
        if head is not None:
            y = jnp.tanh(jnp.dot(y, w2_r[...], preferred_element_type=jnp.float32, precision=PREC)
                         + b2_r[...])
        o_r[...] = y

    row = lambda shp: pl.BlockSpec(shp, lambda i: (i, 0))
    full2 = lambda shp: pl.BlockSpec(shp, lambda i: (0, 0))
    full3 = lambda shp: pl.BlockSpec(shp, lambda i: (0, 0, 0))
    full1 = lambda shp: pl.BlockSpec(shp, lambda i: (0,))
    in_specs = [
        row((B, 128)), row((B, 128)), row((B, 128)), row((B, 128)), row((B, din)),
        row((B, 1)), row((B, 1)), row((B, 1)), row((B, 1)),
        full3((MAXD + 1, din, HID)), full2((MAXD + 1, HID)), full3((MAXD + 1, din, HID)),
        full3((MAXD + 1, din, HID)), full2((MAXD + 1, HID)), full3((MAXD + 1, din, HID)),
        full2((HID, HID)), full1((HID,)),
    ]
    args = [h1a, h1b, h2a, h2b, x, d1a, d1b, d2a, d2b,
            mf1['Wl'], mf1['bl'], mf1['Wr'], mf2['Wl'], mf2['bl'], mf2['Wr'],
            lin['W'], lin['b']]
    if head is not None:
        in_specs += [full2((HID, OUT)), full1((OUT,))]
        args += [head['W'], head['b']]
    return pl.pallas_call(
        body,
        grid=(nblk,),
        in_specs=in_specs,
        out_specs=row((B, dout)),
        out_shape=jax.ShapeDtypeStruct((N, dout), jnp.float32),
        interpret=interpret,
    )(*args)


def _qkv(x, xg, tc, interpret=False):
    din = x.shape[1]
    nblk = N // B
    scale = 1.0 / (HID ** 0.5)

    def body(x_r, xg_r, wq_r, bq_r, wk_r, bk_r, wv_r, bv_r, a_r, v_r):
        q = jnp.dot(xg_r[...], wq_r[...], preferred_element_type=jnp.float32, precision=PREC) + bq_r[...]
        kk = jnp.dot(x_r[...], wk_r[...], preferred_element_type=jnp.float32, precision=PREC) + bk_r[...]
        vv = jnp.dot(x_r[...], wv_r[...], preferred_element_type=jnp.float32, precision=PREC) + bv_r[...]
        a_r[...] = lax.dot_general(kk, q, (((1,), (1,)), ((), ())),
                                   preferred_element_type=jnp.float32,
                                   precision=PREC_F32) * scale
        v_r[...] = vv

    row = lambda shp: pl.BlockSpec(shp, lambda i: (i, 0))
    full2 = lambda shp: pl.BlockSpec(shp, lambda i: (0, 0))
    full1 = lambda shp: pl.BlockSpec(shp, lambda i: (0,))
    return pl.pallas_call(
        body,
        grid=(nblk,),
        in_specs=[row((B, din)), full2((NG, din)),
                  full2((din, HID)), full1((HID,)), full2((din, HID)), full1((HID,)),
                  full2((din, HID)), full1((HID,))],
        out_specs=[row((B, NG)), row((B, HID))],
        out_shape=[jax.ShapeDtypeStruct((N, NG), jnp.float32),
                   jax.ShapeDtypeStruct((N, HID), jnp.float32)],
        interpret=interpret,
    )(x, xg, tc['Wq'], tc['bq'], tc['Wk'], tc['bk'], tc['Wv'], tc['bv'])


def _god_layer(a1, v1, c1a, c1b, a2, v2, c2a, c2b, xg,
               tc1, tc2, lin, head, interpret=False):
    din = xg.shape[1]
    nblk = N // B
    dout = OUT if head is not None else HID

    def body(a1_r, v1_r, c1a_r, c1b_r, a2_r, v2_r, c2a_r, c2b_r, xg_r,
             ws1_r, bs1_r, ws2_r, bs2_r, wlin_r, blin_r, *rest):
        if head is not None:
            w2_r, b2_r, o_r, m1, dn1, g1, m2, dn2, g2 = rest
        else:
            o_r, m1, dn1, g1, m2, dn2, g2 = rest
        j = pl.program_id(0)

        @pl.when(j == 0)
        def _():
            m1[...] = jnp.full((1, NG), -jnp.inf, jnp.float32)
            dn1[...] = jnp.zeros((1, NG), jnp.float32)
            g1[...] = jnp.zeros((HID, NG), jnp.float32)
            m2[...] = jnp.full((1, NG), -jnp.inf, jnp.float32)
            dn2[...] = jnp.zeros((1, NG), jnp.float32)
            g2[...] = jnp.zeros((HID, NG), jnp.float32)

        for a_r, v_r, ca_r, cb_r, m, dn, g in (
                (a1_r, v1_r, c1a_r, c1b_r, m1, dn1, g1),
                (a2_r, v2_r, c2a_r, c2b_r, m2, dn2, g2)):
            av = a_r[...]
            cv = ca_r[...] + cb_r[...]
            mask = cv > 0.0
            am = jnp.where(mask, av, -jnp.inf)
            bm = jnp.max(am, axis=0, keepdims=True)
            m_old = m[...]
            m_new = jnp.maximum(m_old, bm)
            r = jnp.exp(jnp.where(m_new == -jnp.inf, 0.0, m_old - m_new))
            eb = jnp.where(mask, cv * jnp.exp(av - m_new), 0.0)
            dn[...] = dn[...] * r + jnp.sum(eb, axis=0, keepdims=True)
            g[...] = g[...] * r + lax.dot_general(
                v_r[...], eb, (((0,), (0,)), ((), ())),
                preferred_element_type=jnp.float32, precision=PREC_F32)
            m[...] = m_new

        @pl.when(j == nblk - 1)
        def _():
            agg1 = jnp.transpose(g1[...] / jnp.maximum(dn1[...], 1e-16), (1, 0))
            agg2 = jnp.transpose(g2[...] / jnp.maximum(dn2[...], 1e-16), (1, 0))
            xgv = xg_r[...]
            hg = (agg1 + jnp.dot(xgv, ws1_r[...], preferred_element_type=jnp.float32, precision=PREC) + bs1_r[...]
                  + agg2 + jnp.dot(xgv, ws2_r[...], preferred_element_type=jnp.float32, precision=PREC) + bs2_r[...])
            y = jnp.dot(hg, wlin_r[...], preferred_element_type=jnp.float32, precision=PREC) + blin_r[...]
            if head is not None:
                y = jnp.tanh(jnp.dot(y, w2_r[...], preferred_element_type=jnp.float32, precision=PREC)
                             + b2_r[...])
            o_r[...] = y

    row = lambda shp: pl.BlockSpec(shp, lambda i: (i, 0))
    full2 = lambda shp: pl.BlockSpec(shp, lambda i: (0, 0))
    full1 = lambda shp: pl.BlockSpec(shp, lambda i: (0,))
    in_specs = [
        row((B, NG)), row((B, HID)), row((B, NG)), row((B, NG)),
        row((B, NG)), row((B, HID)), row((B, NG)), row((B, NG)),
        full2((NG, din)),
        full2((din, HID)), full1((HID,)), full2((din, HID)), full1((HID,)),
        full2((HID, HID)), full1((HID,)),
    ]
    args = [a1, v1, c1a, c1b, a2, v2, c2a, c2b, xg,
            tc1['Ws'], tc1['bs'], tc2['Ws'], tc2['bs'], lin['W'], lin['b']]
    if head is not None:
        in_specs += [full2((HID, OUT)), full1((OUT,))]
        args += [head['W'], head['b']]
    return pl.pallas_call(
        body,
        grid=(nblk,),
        in_specs=in_specs,
        out_specs=full2((NG, dout)),
        out_shape=jax.ShapeDtypeStruct((NG, dout), jnp.float32),
        scratch_shapes=[pltpu.VMEM((1, NG), jnp.float32),
                        pltpu.VMEM((1, NG), jnp.float32),
                        pltpu.VMEM((HID, NG), jnp.float32)] * 2,
        interpret=interpret,
    )(*args)


def kernel(x_phylonodes_up, x_phylonodes_down, x_godnode,
           edge_index_up_up, edge_index_down_down, edge_index_down_up,
           edge_index_up_down, edge_index_down_god, edge_index_up_god, params):
    z1 = jnp.zeros((NB_DEG // NS,), jnp.float32)
    z1p = jnp.zeros((NB_PAIR // NS,), jnp.float32)
    z128 = jnp.zeros((NP // NS, 128), jnp.float32)

    bc_deg = _build_bincount(NB_DEG, use_pair=False)
    bc_pair = _build_bincount(NB_PAIR, use_pair=True)
    seg1 = _build_segsum(feature_split=False)
    seg2 = _build_segsum(feature_split=True)

    eis = {
        'uu': edge_index_up_up, 'dd': edge_index_down_down,
        'du': edge_index_down_up, 'ud': edge_index_up_down,
        'dg': edge_index_down_god, 'ug': edge_index_up_god,
    }
    src = {k: v[0] for k, v in eis.items()}
    dst = {k: v[1] for k, v in eis.items()}

    deg = {}
    for t in ('uu', 'du', 'dd', 'ud'):
        cnt = bc_deg(src[t], dst[t], z1).reshape(NC, NB_DEG)
        deg[t] = (cnt[0, :N].reshape(N, 1), cnt[1, :N].reshape(N, 1))
    cmat = {}
    for t in ('dg', 'ug'):
        cnt = bc_pair(src[t], dst[t], z1p).reshape(NC, NB_PAIR)
        cc = cnt[:, :N * NG].reshape(NC, N, NG)
        cmat[t] = (cc[0], cc[1])

    x_up, x_down, x_god = x_phylonodes_up, x_phylonodes_down, x_godnode
    for li, lp in enumerate(params['layers']):
        din = x_up.shape[1]
        if din == DF:
            seg = seg1
            xu = (x_up, x_up)
            xd = (x_down, x_down)
        else:
            seg = seg2
            xu = (x_up[:, :128], x_up[:, 128:])
            xd = (x_down[:, :128], x_down[:, 128:])
        h = {}
        for t, xs in (('uu', xu), ('du', xd), ('dd', xd), ('ud', xu)):
            hh = seg(xs[0], xs[1], src[t], dst[t], z128)
            h[t] = (hh[0, :N], hh[1, :N])
        final = li == len(params['layers']) - 1
        head_up = params['lins2']['up'] if final else None
        head_down = params['lins2']['down'] if final else None
        head_god = params['lins2']['god'] if final else None
        new_up = _mf_layer(h['uu'][0], h['uu'][1], h['du'][0], h['du'][1], x_up,
                           deg['uu'][0], deg['uu'][1], deg['du'][0], deg['du'][1],
                           lp['mf_uu'], lp['mf_du'], lp['lin_up'], head_up)
        new_down = _mf_layer(h['dd'][0], h['dd'][1], h['ud'][0], h['ud'][1], x_down,
                             deg['dd'][0], deg['dd'][1], deg['ud'][0], deg['ud'][1],
                             lp['mf_dd'], lp['mf_ud'], lp['lin_down'], head_down)
        a1, v1 = _qkv(x_down, x_god, lp['tc_dg'])
        a2, v2 = _qkv(x_up, x_god, lp['tc_ug'])
        new_god = _god_layer(a1, v1, cmat['dg'][0], cmat['dg'][1],
                             a2, v2, cmat['ug'][0], cmat['ug'][1], x_god,
                             lp['tc_dg'], lp['tc_ug'], lp['lin_god'], head_god)
        x_up, x_down, x_god = new_up, new_down, new_god
    return (x_up, x_down, x_god)


# speedup vs baseline: 10.0170x; 1.3821x over previous
import functools

import jax
import jax.numpy as jnp
from jax import lax
from jax.experimental import pallas as pl
from jax.experimental.pallas import tpu as pltpu
from jax.experimental.pallas import tpu_sc as plsc

N = 10000
NP = 10240
NG = 64
DF = 128
HID = 256
OUT = 128
E = 320000
MAXD = 10
NC = 2
NS = 16
CHUNK = 128
NB_DEG = NP
NB_PAIR = 655360
B = 400
PREC = jax.lax.Precision.DEFAULT
PREC_F32 = jax.lax.Precision.HIGHEST


def _sc_mesh():
    return plsc.VectorSubcoreMesh(
        core_axis_name="c", subcore_axis_name="s", num_cores=NC, num_subcores=NS)


def _build_bincount(nbins, use_pair):
    slab = nbins // NS
    nchunks = E // CHUNK
    ntiles = NC * NS
    iters = (nchunks + ntiles - 1) // ntiles
    assert slab % CHUNK == 0

    def body(src_hbm, dst_hbm, zeros_hbm, out_hbm, src_v, dst_v, bin_v, ones_v, acc, sem):
        del sem
        cid = lax.axis_index("c")
        sid = lax.axis_index("s")
        tid = cid * NS + sid
        pltpu.sync_copy(zeros_hbm, acc.at[pl.ds(sid * slab, slab)])
        for k in range(CHUNK // 16):
            ones_v[pl.ds(k * 16, 16)] = jnp.full((16,), 1.0, jnp.float32)
        plsc.subcore_barrier()

        def chunk_body(i, carry):
            chunk = tid + i * ntiles

            @pl.when(chunk < nchunks)
            def _():
                off = chunk * CHUNK
                pltpu.sync_copy(dst_hbm.at[pl.ds(off, CHUNK)], dst_v)
                if use_pair:
                    pltpu.sync_copy(src_hbm.at[pl.ds(off, CHUNK)], src_v)
                    for k in range(CHUNK // 16):
                        s = src_v[pl.ds(k * 16, 16)]
                        d = dst_v[pl.ds(k * 16, 16)]
                        bin_v[pl.ds(k * 16, 16)] = s * NG + d
                    pltpu.sync_copy(ones_v, acc.at[bin_v], add=True)
                else:
                    pltpu.sync_copy(ones_v, acc.at[dst_v], add=True)

            return carry

        lax.fori_loop(0, iters, chunk_body, 0)
        plsc.subcore_barrier()
        pltpu.sync_copy(acc.at[pl.ds(sid * slab, slab)],
                        out_hbm.at[pl.ds(cid * nbins + sid * slab, slab)])

    return pl.kernel(
        body,
        out_type=jax.ShapeDtypeStruct((NC * nbins,), jnp.float32),
        mesh=_sc_mesh(),
        scratch_types=[
            pltpu.VMEM((CHUNK,), jnp.int32),
            pltpu.VMEM((CHUNK,), jnp.int32),
            pltpu.VMEM((CHUNK,), jnp.int32),
            pltpu.VMEM((CHUNK,), jnp.float32),
            pltpu.VMEM_SHARED((nbins,), jnp.float32),
            pltpu.SemaphoreType.DMA,
        ],
    )


def _build_segsum(feature_split, nrows=N, nedges=E):
    W = 128
    rows = NP // NS
    nchunks = nedges // CHUNK
    ntiles = NS if feature_split else NC * NS
    iters = (nchunks + ntiles - 1) // ntiles

    def body(xa, xb, src_hbm, dst_hbm, zeros_hbm, out_hbm,
             src_v, dst_v, rows_v, acc, isem, gsem, ssem):
        cid = lax.axis_index("c")
        sid = lax.axis_index("s")
        tix = sid if feature_split else cid * NS + sid
        r0 = sid * rows
        pltpu.sync_copy(zeros_hbm, acc.at[pl.ds(r0, rows), :])
        plsc.subcore_barrier()
        n_in = (nchunks - tix + ntiles - 1) // ntiles

        def start_idx(i, b):
            off = (tix + i * ntiles) * CHUNK
            pltpu.async_copy(src_hbm.at[pl.ds(off, CHUNK)], src_v.at[b], isem)
            pltpu.async_copy(dst_hbm.at[pl.ds(off, CHUNK)], dst_v.at[b], isem)

        def wait_idx(b):
            pltpu.make_async_copy(src_hbm.at[pl.ds(0, CHUNK)], src_v.at[b], isem).wait()
            pltpu.make_async_copy(dst_hbm.at[pl.ds(0, CHUNK)], dst_v.at[b], isem).wait()

        def wait_scatter(b):
            pltpu.make_async_copy(rows_v.at[b], acc.at[dst_v.at[b]], ssem).wait()

        @pl.when(tix < nchunks)
        def _():
            start_idx(0, 0)

        def chunk_body(i, carry):
            b = lax.rem(i, 2)
            nb = lax.rem(i + 1, 2)

            @pl.when(tix + i * ntiles < nchunks)
            def _():
                wait_idx(b)

                @pl.when(cid == 0)
                def _():
                    pltpu.async_copy(xa.at[src_v.at[b]], rows_v.at[b], gsem).wait()

                @pl.when(cid == 1)
                def _():
                    pltpu.async_copy(xb.at[src_v.at[b]], rows_v.at[b], gsem).wait()

                @pl.when(i >= 1)
                def _():
                    wait_scatter(nb)

                pltpu.async_copy(rows_v.at[b], acc.at[dst_v.at[b]], ssem, add=True)

                @pl.when(tix + (i + 1) * ntiles < nchunks)
                def _():
                    start_idx(i + 1, nb)

            return carry

        lax.fori_loop(0, iters, chunk_body, 0)

        @pl.when(n_in >= 1)
        def _():
            wait_scatter(lax.rem(n_in - 1, 2))

        plsc.subcore_barrier()
        pltpu.sync_copy(acc.at[pl.ds(r0, rows), :],
                        out_hbm.at[cid, pl.ds(r0, rows), :])

    return pl.kernel(
        body,
        out_type=jax.ShapeDtypeStruct((NC, NP, W), jnp.float32),
        mesh=_sc_mesh(),
        scratch_types=[
            pltpu.VMEM((2, CHUNK), jnp.int32),
            pltpu.VMEM((2, CHUNK), jnp.int32),
            pltpu.VMEM((2, CHUNK, W), jnp.float32),
            pltpu.VMEM_SHARED((NP, W), jnp.float32),
            pltpu.SemaphoreType.DMA,
            pltpu.SemaphoreType.DMA,
            pltpu.SemaphoreType.DMA,
        ],
    )

def _mf_layer(h1a, h1b, h2a, h2b, x, d1a, d1b, d2a, d2b,
              mf1, mf2, lin, head, interpret=False):
    din = x.shape[1]
    concat = din > 128
    nblk = N // B
    dout = OUT if head is not None else HID

    def body(h1a_r, h1b_r, h2a_r, h2b_r, x_r, d1a_r, d1b_r, d2a_r, d2b_r,
             wl1_r, bl1_r, wr1_r, wl2_r, bl2_r, wr2_r,
             wlin_r, blin_r, *rest):
        if head is not None:
            w2_r, b2_r, o_r = rest
        else:
            (o_r,) = rest
        if concat:
            h1 = jnp.concatenate([h1a_r[...], h1b_r[...]], axis=1)
            h2 = jnp.concatenate([h2a_r[...], h2b_r[...]], axis=1)
        else:
            h1 = h1a_r[...] + h1b_r[...]
            h2 = h2a_r[...] + h2b_r[...]
        xv = x_r[...]
        deg1 = jnp.clip(d1a_r[...] + d1b_r[...], 0.0, float(MAXD)).astype(jnp.int32)
        deg2 = jnp.clip(d2a_r[...] + d2b_r[...], 0.0, float(MAXD)).astype(jnp.int32)
        iota = lax.broadcasted_iota(jnp.int32, (B, MAXD + 1), 1)
        oh1 = (deg1 == iota).astype(jnp.float32)
        oh2 = (deg2 == iota).astype(jnp.float32)
        acc = jnp.zeros((B, HID), jnp.float32)
        for d in range(MAXD + 1):
            t1 = (jnp.dot(h1, wl1_r[d], preferred_element_type=jnp.float32, precision=PREC)
                  + bl1_r[d]
                  + jnp.dot(xv, wr1_r[d], preferred_element_type=jnp.float32, precision=PREC))
            t2 = (jnp.dot(h2, wl2_r[d], preferred_element_type=jnp.float32, precision=PREC)
                  + bl2_r[d]
                  + jnp.dot(xv, wr2_r[d], preferred_element_type=jnp.float32, precision=PREC))
            t1 = t1.astype(jnp.bfloat16).astype(jnp.float32)
            t2 = t2.astype(jnp.bfloat16).astype(jnp.float32)
            acc = acc + oh1[:, d:d + 1] * t1 + oh2[:, d:d + 1] * t2
        y = jnp.dot(acc, wlin_r[...], preferred_element_type=jnp.float32, precision=PREC) + blin_r[...]
        if head is not None:
            y = jnp.tanh(jnp.dot(y, w2_r[...], preferred_element_type=jnp.float32, precision=PREC)
                         + b2_r[...])
        o_r[...] = y

    row = lambda shp: pl.BlockSpec(shp, lambda i: (i, 0))
    full2 = lambda shp: pl.BlockSpec(shp, lambda i: (0, 0))
    full3 = lambda shp: pl.BlockSpec(shp, lambda i: (0, 0, 0))
    full1 = lambda shp: pl.BlockSpec(shp, lambda i: (0,))
    in_specs = [
        row((B, 128)), row((B, 128)), row((B, 128)), row((B, 128)), row((B, din)),
        row((B, 1)), row((B, 1)), row((B, 1)), row((B, 1)),
        full3((MAXD + 1, din, HID)), full2((MAXD + 1, HID)), full3((MAXD + 1, din, HID)),
        full3((MAXD + 1, din, HID)), full2((MAXD + 1, HID)), full3((MAXD + 1, din, HID)),
        full2((HID, HID)), full1((HID,)),
    ]
    args = [h1a, h1b, h2a, h2b, x, d1a, d1b, d2a, d2b,
            mf1['Wl'], mf1['bl'], mf1['Wr'], mf2['Wl'], mf2['bl'], mf2['Wr'],
            lin['W'], lin['b']]
    if head is not None:
        in_specs += [full2((HID, OUT)), full1((OUT,))]
        args += [head['W'], head['b']]
    return pl.pallas_call(
        body,
        grid=(nblk,),
        in_specs=in_specs,
        out_specs=row((B, dout)),
        out_shape=jax.ShapeDtypeStruct((N, dout), jnp.float32),
        interpret=interpret,
    )(*args)


def _qkv(x, xg, tc, interpret=False):
    din = x.shape[1]
    nblk = N // B
    scale = 1.0 / (HID ** 0.5)

    def body(x_r, xg_r, wq_r, bq_r, wk_r, bk_r, wv_r, bv_r, a_r, v_r):
        q = jnp.dot(xg_r[...], wq_r[...], preferred_element_type=jnp.float32, precision=PREC) + bq_r[...]
        kk = jnp.dot(x_r[...], wk_r[...], preferred_element_type=jnp.float32, precision=PREC) + bk_r[...]
        vv = jnp.dot(x_r[...], wv_r[...], preferred_element_type=jnp.float32, precision=PREC) + bv_r[...]
        a_r[...] = lax.dot_general(kk, q, (((1,), (1,)), ((), ())),
                                   preferred_element_type=jnp.float32,
                                   precision=PREC_F32) * scale
        v_r[...] = vv

    row = lambda shp: pl.BlockSpec(shp, lambda i: (i, 0))
    full2 = lambda shp: pl.BlockSpec(shp, lambda i: (0, 0))
    full1 = lambda shp: pl.BlockSpec(shp, lambda i: (0,))
    return pl.pallas_call(
        body,
        grid=(nblk,),
        in_specs=[row((B, din)), full2((NG, din)),
                  full2((din, HID)), full1((HID,)), full2((din, HID)), full1((HID,)),
                  full2((din, HID)), full1((HID,))],
        out_specs=[row((B, NG)), row((B, HID))],
        out_shape=[jax.ShapeDtypeStruct((N, NG), jnp.float32),
                   jax.ShapeDtypeStruct((N, HID), jnp.float32)],
        interpret=interpret,
    )(x, xg, tc['Wq'], tc['bq'], tc['Wk'], tc['bk'], tc['Wv'], tc['bv'])


def _god_layer(a1, v1, c1a, c1b, a2, v2, c2a, c2b, xg,
               tc1, tc2, lin, head, interpret=False):
    din = xg.shape[1]
    nblk = N // B
    dout = OUT if head is not None else HID

    def body(a1_r, v1_r, c1a_r, c1b_r, a2_r, v2_r, c2a_r, c2b_r, xg_r,
             ws1_r, bs1_r, ws2_r, bs2_r, wlin_r, blin_r, *rest):
        if head is not None:
            w2_r, b2_r, o_r, m1, dn1, g1, m2, dn2, g2 = rest
        else:
            o_r, m1, dn1, g1, m2, dn2, g2 = rest
        j = pl.program_id(0)

        @pl.when(j == 0)
        def _():
            m1[...] = jnp.full((1, NG), -jnp.inf, jnp.float32)
            dn1[...] = jnp.zeros((1, NG), jnp.float32)
            g1[...] = jnp.zeros((HID, NG), jnp.float32)
            m2[...] = jnp.full((1, NG), -jnp.inf, jnp.float32)
            dn2[...] = jnp.zeros((1, NG), jnp.float32)
            g2[...] = jnp.zeros((HID, NG), jnp.float32)

        for a_r, v_r, ca_r, cb_r, m, dn, g in (
                (a1_r, v1_r, c1a_r, c1b_r, m1, dn1, g1),
                (a2_r, v2_r, c2a_r, c2b_r, m2, dn2, g2)):
            av = a_r[...]
            cv = ca_r[...] + cb_r[...]
            mask = cv > 0.0
            am = jnp.where(mask, av, -jnp.inf)
            bm = jnp.max(am, axis=0, keepdims=True)
            m_old = m[...]
            m_new = jnp.maximum(m_old, bm)
            r = jnp.exp(jnp.where(m_new == -jnp.inf, 0.0, m_old - m_new))
            eb = jnp.where(mask, cv * jnp.exp(av - m_new), 0.0)
            dn[...] = dn[...] * r + jnp.sum(eb, axis=0, keepdims=True)
            g[...] = g[...] * r + lax.dot_general(
                v_r[...], eb, (((0,), (0,)), ((), ())),
                preferred_element_type=jnp.float32, precision=PREC_F32)
            m[...] = m_new

        @pl.when(j == nblk - 1)
        def _():
            agg1 = jnp.transpose(g1[...] / jnp.maximum(dn1[...], 1e-16), (1, 0))
            agg2 = jnp.transpose(g2[...] / jnp.maximum(dn2[...], 1e-16), (1, 0))
            xgv = xg_r[...]
            hg = (agg1 + jnp.dot(xgv, ws1_r[...], preferred_element_type=jnp.float32, precision=PREC) + bs1_r[...]
                  + agg2 + jnp.dot(xgv, ws2_r[...], preferred_element_type=jnp.float32, precision=PREC) + bs2_r[...])
            y = jnp.dot(hg, wlin_r[...], preferred_element_type=jnp.float32, precision=PREC) + blin_r[...]
            if head is not None:
                y = jnp.tanh(jnp.dot(y, w2_r[...], preferred_element_type=jnp.float32, precision=PREC)
                             + b2_r[...])
            o_r[...] = y

    row = lambda shp: pl.BlockSpec(shp, lambda i: (i, 0))
    full2 = lambda shp: pl.BlockSpec(shp, lambda i: (0, 0))
    full1 = lambda shp: pl.BlockSpec(shp, lambda i: (0,))
    in_specs = [
        row((B, NG)), row((B, HID)), row((B, NG)), row((B, NG)),
        row((B, NG)), row((B, HID)), row((B, NG)), row((B, NG)),
        full2((NG, din)),
        full2((din, HID)), full1((HID,)), full2((din, HID)), full1((HID,)),
        full2((HID, HID)), full1((HID,)),
    ]
    args = [a1, v1, c1a, c1b, a2, v2, c2a, c2b, xg,
            tc1['Ws'], tc1['bs'], tc2['Ws'], tc2['bs'], lin['W'], lin['b']]
    if head is not None:
        in_specs += [full2((HID, OUT)), full1((OUT,))]
        args += [head['W'], head['b']]
    return pl.pallas_call(
        body,
        grid=(nblk,),
        in_specs=in_specs,
        out_specs=full2((NG, dout)),
        out_shape=jax.ShapeDtypeStruct((NG, dout), jnp.float32),
        scratch_shapes=[pltpu.VMEM((1, NG), jnp.float32),
                        pltpu.VMEM((1, NG), jnp.float32),
                        pltpu.VMEM((HID, NG), jnp.float32)] * 2,
        interpret=interpret,
    )(*args)


def kernel(x_phylonodes_up, x_phylonodes_down, x_godnode,
           edge_index_up_up, edge_index_down_down, edge_index_down_up,
           edge_index_up_down, edge_index_down_god, edge_index_up_god, params):
    z1 = jnp.zeros((NB_DEG // NS,), jnp.float32)
    z1p = jnp.zeros((NB_PAIR // NS,), jnp.float32)
    z128 = jnp.zeros((NP // NS, 128), jnp.float32)

    bc_deg = _build_bincount(NB_DEG, use_pair=False)
    bc_pair = _build_bincount(NB_PAIR, use_pair=True)
    seg1 = _build_segsum(feature_split=False)
    seg2 = _build_segsum(feature_split=True)

    eis = {
        'uu': edge_index_up_up, 'dd': edge_index_down_down,
        'du': edge_index_down_up, 'ud': edge_index_up_down,
        'dg': edge_index_down_god, 'ug': edge_index_up_god,
    }
    src = {k: v[0] for k, v in eis.items()}
    dst = {k: v[1] for k, v in eis.items()}

    deg = {}
    for t in ('uu', 'du', 'dd', 'ud'):
        cnt = bc_deg(src[t], dst[t], z1).reshape(NC, NB_DEG)
        deg[t] = (cnt[0, :N].reshape(N, 1), cnt[1, :N].reshape(N, 1))
    cmat = {}
    for t in ('dg', 'ug'):
        cnt = bc_pair(src[t], dst[t], z1p).reshape(NC, NB_PAIR)
        cc = cnt[:, :N * NG].reshape(NC, N, NG)
        cmat[t] = (cc[0], cc[1])

    x_up, x_down, x_god = x_phylonodes_up, x_phylonodes_down, x_godnode
    for li, lp in enumerate(params['layers']):
        din = x_up.shape[1]
        if din == DF:
            seg = seg1
            xu = (x_up, x_up)
            xd = (x_down, x_down)
        else:
            seg = seg2
            xu = (x_up[:, :128], x_up[:, 128:])
            xd = (x_down[:, :128], x_down[:, 128:])
        h = {}
        for t, xs in (('uu', xu), ('du', xd), ('dd', xd), ('ud', xu)):
            hh = seg(xs[0], xs[1], src[t], dst[t], z128)
            h[t] = (hh[0, :N], hh[1, :N])
        final = li == len(params['layers']) - 1
        head_up = params['lins2']['up'] if final else None
        head_down = params['lins2']['down'] if final else None
        head_god = params['lins2']['god'] if final else None
        new_up = _mf_layer(h['uu'][0], h['uu'][1], h['du'][0], h['du'][1], x_up,
                           deg['uu'][0], deg['uu'][1], deg['du'][0], deg['du'][1],
                           lp['mf_uu'], lp['mf_du'], lp['lin_up'], head_up)
        new_down = _mf_layer(h['dd'][0], h['dd'][1], h['ud'][0], h['ud'][1], x_down,
                             deg['dd'][0], deg['dd'][1], deg['ud'][0], deg['ud'][1],
                             lp['mf_dd'], lp['mf_ud'], lp['lin_down'], head_down)
        a1, v1 = _qkv(x_down, x_god, lp['tc_dg'])
        a2, v2 = _qkv(x_up, x_god, lp['tc_ug'])
        new_god = _god_layer(a1, v1, cmat['dg'][0], cmat['dg'][1],
                             a2, v2, cmat['ug'][0], cmat['ug'][1], x_god,
                             lp['tc_dg'], lp['tc_ug'], lp['lin_god'], head_god)
        x_up, x_down, x_god = new_up, new_down, new_god
    return (x_up, x_down, x_god)


# speedup vs baseline: 10.4364x; 1.0419x over previous
import functools

import jax
import jax.numpy as jnp
from jax import lax
from jax.experimental import pallas as pl
from jax.experimental.pallas import tpu as pltpu
from jax.experimental.pallas import tpu_sc as plsc

N = 10000
NP = 10240
NG = 64
DF = 128
HID = 256
OUT = 128
E = 320000
MAXD = 10
NC = 2
NS = 16
CHUNK = 128
NB_DEG = NP
NB_PAIR = 655360
B = 400
PREC = jax.lax.Precision.DEFAULT
PREC_F32 = jax.lax.Precision.HIGHEST


def _sc_mesh():
    return plsc.VectorSubcoreMesh(
        core_axis_name="c", subcore_axis_name="s", num_cores=NC, num_subcores=NS)


def _build_bincount(nbins, use_pair):
    slab = nbins // NS
    nchunks = E // CHUNK
    ntiles = NC * NS
    iters = (nchunks + ntiles - 1) // ntiles
    assert slab % CHUNK == 0

    def body(src_hbm, dst_hbm, zeros_hbm, out_hbm, src_v, dst_v, bin_v, ones_v,
             acc, isem, ssem):
        cid = lax.axis_index("c")
        sid = lax.axis_index("s")
        tid = cid * NS + sid
        pltpu.sync_copy(zeros_hbm, acc.at[pl.ds(sid * slab, slab)])
        for k in range(CHUNK // 16):
            ones_v[pl.ds(k * 16, 16)] = jnp.full((16,), 1.0, jnp.float32)
        plsc.subcore_barrier()
        n_in = (nchunks - tid + ntiles - 1) // ntiles

        def start_idx(i, b):
            off = (tid + i * ntiles) * CHUNK
            pltpu.async_copy(dst_hbm.at[pl.ds(off, CHUNK)], dst_v.at[b], isem)
            if use_pair:
                pltpu.async_copy(src_hbm.at[pl.ds(off, CHUNK)], src_v.at[b], isem)

        def wait_idx(b):
            pltpu.make_async_copy(dst_hbm.at[pl.ds(0, CHUNK)], dst_v.at[b], isem).wait()
            if use_pair:
                pltpu.make_async_copy(src_hbm.at[pl.ds(0, CHUNK)], src_v.at[b], isem).wait()

        def idx_ref(b):
            return bin_v.at[b] if use_pair else dst_v.at[b]

        def wait_scatter(b):
            pltpu.make_async_copy(ones_v, acc.at[idx_ref(b)], ssem).wait()

        @pl.when(tid < nchunks)
        def _():
            start_idx(0, 0)

        def chunk_body(i, carry):
            b = lax.rem(i, 2)
            nb = lax.rem(i + 1, 2)

            @pl.when(tid + i * ntiles < nchunks)
            def _():
                wait_idx(b)
                if use_pair:
                    for k in range(CHUNK // 16):
                        s = src_v[b, pl.ds(k * 16, 16)]
                        d = dst_v[b, pl.ds(k * 16, 16)]
                        bin_v[b, pl.ds(k * 16, 16)] = s * NG + d

                @pl.when(i >= 1)
                def _():
                    wait_scatter(nb)

                pltpu.async_copy(ones_v, acc.at[idx_ref(b)], ssem, add=True)

                @pl.when(tid + (i + 1) * ntiles < nchunks)
                def _():
                    start_idx(i + 1, nb)

            return carry

        lax.fori_loop(0, iters, chunk_body, 0)

        @pl.when(n_in >= 1)
        def _():
            wait_scatter(lax.rem(n_in - 1, 2))

        plsc.subcore_barrier()
        pltpu.sync_copy(acc.at[pl.ds(sid * slab, slab)],
                        out_hbm.at[pl.ds(cid * nbins + sid * slab, slab)])

    return pl.kernel(
        body,
        out_type=jax.ShapeDtypeStruct((NC * nbins,), jnp.float32),
        mesh=_sc_mesh(),
        scratch_types=[
            pltpu.VMEM((2, CHUNK), jnp.int32),
            pltpu.VMEM((2, CHUNK), jnp.int32),
            pltpu.VMEM((2, CHUNK), jnp.int32),
            pltpu.VMEM((CHUNK,), jnp.float32),
            pltpu.VMEM_SHARED((nbins,), jnp.float32),
            pltpu.SemaphoreType.DMA,
            pltpu.SemaphoreType.DMA,
        ],
    )


def _build_segsum(feature_split, nrows=N, nedges=E):
    W = 128
    rows = NP // NS
    nchunks = nedges // CHUNK
    ntiles = NS if feature_split else NC * NS
    iters = (nchunks + ntiles - 1) // ntiles

    def body(xa, xb, src_hbm, dst_hbm, zeros_hbm, out_hbm,
             src_v, dst_v, rows_v, acc, isem, gsem, ssem):
        cid = lax.axis_index("c")
        sid = lax.axis_index("s")
        tix = sid if feature_split else cid * NS + sid
        r0 = sid * rows
        pltpu.sync_copy(zeros_hbm, acc.at[pl.ds(r0, rows), :])
        plsc.subcore_barrier()
        n_in = (nchunks - tix + ntiles - 1) // ntiles

        def start_idx(i, b):
            off = (tix + i * ntiles) * CHUNK
            pltpu.async_copy(src_hbm.at[pl.ds(off, CHUNK)], src_v.at[b], isem)
            pltpu.async_copy(dst_hbm.at[pl.ds(off, CHUNK)], dst_v.at[b], isem)

        def wait_idx(b):
            pltpu.make_async_copy(src_hbm.at[pl.ds(0, CHUNK)], src_v.at[b], isem).wait()
            pltpu.make_async_copy(dst_hbm.at[pl.ds(0, CHUNK)], dst_v.at[b], isem).wait()

        def wait_scatter(b):
            pltpu.make_async_copy(rows_v.at[b], acc.at[dst_v.at[b]], ssem).wait()

        @pl.when(tix < nchunks)
        def _():
            start_idx(0, 0)

        def chunk_body(i, carry):
            b = lax.rem(i, 2)
            nb = lax.rem(i + 1, 2)

            @pl.when(tix + i * ntiles < nchunks)
            def _():
                wait_idx(b)

                @pl.when(cid == 0)
                def _():
                    pltpu.async_copy(xa.at[src_v.at[b]], rows_v.at[b], gsem).wait()

                @pl.when(cid == 1)
                def _():
                    pltpu.async_copy(xb.at[src_v.at[b]], rows_v.at[b], gsem).wait()

                @pl.when(i >= 1)
                def _():
                    wait_scatter(nb)

                pltpu.async_copy(rows_v.at[b], acc.at[dst_v.at[b]], ssem, add=True)

                @pl.when(tix + (i + 1) * ntiles < nchunks)
                def _():
                    start_idx(i + 1, nb)

            return carry

        lax.fori_loop(0, iters, chunk_body, 0)

        @pl.when(n_in >= 1)
        def _():
            wait_scatter(lax.rem(n_in - 1, 2))

        plsc.subcore_barrier()
        pltpu.sync_copy(acc.at[pl.ds(r0, rows), :],
                        out_hbm.at[cid, pl.ds(r0, rows), :])

    return pl.kernel(
        body,
        out_type=jax.ShapeDtypeStruct((NC, NP, W), jnp.float32),
        mesh=_sc_mesh(),
        scratch_types=[
            pltpu.VMEM((2, CHUNK), jnp.int32),
            pltpu.VMEM((2, CHUNK), jnp.int32),
            pltpu.VMEM((2, CHUNK, W), jnp.float32),
            pltpu.VMEM_SHARED((NP, W), jnp.float32),
            pltpu.SemaphoreType.DMA,
            pltpu.SemaphoreType.DMA,
            pltpu.SemaphoreType.DMA,
        ],
    )

def _mf_layer(h1a, h1b, h2a, h2b, x, d1a, d1b, d2a, d2b,
              mf1, mf2, lin, head, interpret=False):
    din = x.shape[1]
    concat = din > 128
    nblk = N // B
    dout = OUT if head is not None else HID

    def body(h1a_r, h1b_r, h2a_r, h2b_r, x_r, d1a_r, d1b_r, d2a_r, d2b_r,
             wl1_r, bl1_r, wr1_r, wl2_r, bl2_r, wr2_r,
             wlin_r, blin_r, *rest):
        if head is not None:
            w2_r, b2_r, o_r = rest
        else:
            (o_r,) = rest
        if concat:
            h1 = jnp.concatenate([h1a_r[...], h1b_r[...]], axis=1)
            h2 = jnp.concatenate([h2a_r[...], h2b_r[...]], axis=1)
        else:
            h1 = h1a_r[...] + h1b_r[...]
            h2 = h2a_r[...] + h2b_r[...]
        xv = x_r[...]
        deg1 = jnp.clip(d1a_r[...] + d1b_r[...], 0.0, float(MAXD)).astype(jnp.int32)
        deg2 = jnp.clip(d2a_r[...] + d2b_r[...], 0.0, float(MAXD)).astype(jnp.int32)
        iota = lax.broadcasted_iota(jnp.int32, (B, MAXD + 1), 1)
        oh1 = (deg1 == iota).astype(jnp.float32)
        oh2 = (deg2 == iota).astype(jnp.float32)
        acc = jnp.zeros((B, HID), jnp.float32)
        for d in range(MAXD + 1):
            t1 = (jnp.dot(h1, wl1_r[d], preferred_element_type=jnp.float32, precision=PREC)
                  + bl1_r[d]
                  + jnp.dot(xv, wr1_r[d], preferred_element_type=jnp.float32, precision=PREC))
            t2 = (jnp.dot(h2, wl2_r[d], preferred_element_type=jnp.float32, precision=PREC)
                  + bl2_r[d]
                  + jnp.dot(xv, wr2_r[d], preferred_element_type=jnp.float32, precision=PREC))
            t1 = t1.astype(jnp.bfloat16).astype(jnp.float32)
            t2 = t2.astype(jnp.bfloat16).astype(jnp.float32)
            acc = acc + oh1[:, d:d + 1] * t1 + oh2[:, d:d + 1] * t2
        y = jnp.dot(acc, wlin_r[...], preferred_element_type=jnp.float32, precision=PREC) + blin_r[...]
        if head is not None:
            y = jnp.tanh(jnp.dot(y, w2_r[...], preferred_element_type=jnp.float32, precision=PREC)
                         + b2_r[...])
        o_r[...] = y

    row = lambda shp: pl.BlockSpec(shp, lambda i: (i, 0))
    full2 = lambda shp: pl.BlockSpec(shp, lambda i: (0, 0))
    full3 = lambda shp: pl.BlockSpec(shp, lambda i: (0, 0, 0))
    full1 = lambda shp: pl.BlockSpec(shp, lambda i: (0,))
    in_specs = [
        row((B, 128)), row((B, 128)), row((B, 128)), row((B, 128)), row((B, din)),
        row((B, 1)), row((B, 1)), row((B, 1)), row((B, 1)),
        full3((MAXD + 1, din, HID)), full2((MAXD + 1, HID)), full3((MAXD + 1, din, HID)),
        full3((MAXD + 1, din, HID)), full2((MAXD + 1, HID)), full3((MAXD + 1, din, HID)),
        full2((HID, HID)), full1((HID,)),
    ]
    args = [h1a, h1b, h2a, h2b, x, d1a, d1b, d2a, d2b,
            mf1['Wl'], mf1['bl'], mf1['Wr'], mf2['Wl'], mf2['bl'], mf2['Wr'],
            lin['W'], lin['b']]
    if head is not None:
        in_specs += [full2((HID, OUT)), full1((OUT,))]
        args += [head['W'], head['b']]
    return pl.pallas_call(
        body,
        grid=(nblk,),
        in_specs=in_specs,
        out_specs=row((B, dout)),
        out_shape=jax.ShapeDtypeStruct((N, dout), jnp.float32),
        interpret=interpret,
    )(*args)


def _qkv(x, xg, tc, interpret=False):
    din = x.shape[1]
    nblk = N // B
    scale = 1.0 / (HID ** 0.5)

    def body(x_r, xg_r, wq_r, bq_r, wk_r, bk_r, wv_r, bv_r, a_r, v_r):
        q = jnp.dot(xg_r[...], wq_r[...], preferred_element_type=jnp.float32, precision=PREC) + bq_r[...]
        kk = jnp.dot(x_r[...], wk_r[...], preferred_element_type=jnp.float32, precision=PREC) + bk_r[...]
        vv = jnp.dot(x_r[...], wv_r[...], preferred_element_type=jnp.float32, precision=PREC) + bv_r[...]
        a_r[...] = lax.dot_general(kk, q, (((1,), (1,)), ((), ())),
                                   preferred_element_type=jnp.float32,
                                   precision=PREC_F32) * scale
        v_r[...] = vv

    row = lambda shp: pl.BlockSpec(shp, lambda i: (i, 0))
    full2 = lambda shp: pl.BlockSpec(shp, lambda i: (0, 0))
    full1 = lambda shp: pl.BlockSpec(shp, lambda i: (0,))
    return pl.pallas_call(
        body,
        grid=(nblk,),
        in_specs=[row((B, din)), full2((NG, din)),
                  full2((din, HID)), full1((HID,)), full2((din, HID)), full1((HID,)),
                  full2((din, HID)), full1((HID,))],
        out_specs=[row((B, NG)), row((B, HID))],
        out_shape=[jax.ShapeDtypeStruct((N, NG), jnp.float32),
                   jax.ShapeDtypeStruct((N, HID), jnp.float32)],
        interpret=interpret,
    )(x, xg, tc['Wq'], tc['bq'], tc['Wk'], tc['bk'], tc['Wv'], tc['bv'])


def _god_layer(a1, v1, c1a, c1b, a2, v2, c2a, c2b, xg,
               tc1, tc2, lin, head, interpret=False):
    din = xg.shape[1]
    nblk = N // B
    dout = OUT if head is not None else HID

    def body(a1_r, v1_r, c1a_r, c1b_r, a2_r, v2_r, c2a_r, c2b_r, xg_r,
             ws1_r, bs1_r, ws2_r, bs2_r, wlin_r, blin_r, *rest):
        if head is not None:
            w2_r, b2_r, o_r, m1, dn1, g1, m2, dn2, g2 = rest
        else:
            o_r, m1, dn1, g1, m2, dn2, g2 = rest
        j = pl.program_id(0)

        @pl.when(j == 0)
        def _():
            m1[...] = jnp.full((1, NG), -jnp.inf, jnp.float32)
            dn1[...] = jnp.zeros((1, NG), jnp.float32)
            g1[...] = jnp.zeros((HID, NG), jnp.float32)
            m2[...] = jnp.full((1, NG), -jnp.inf, jnp.float32)
            dn2[...] = jnp.zeros((1, NG), jnp.float32)
            g2[...] = jnp.zeros((HID, NG), jnp.float32)

        for a_r, v_r, ca_r, cb_r, m, dn, g in (
                (a1_r, v1_r, c1a_r, c1b_r, m1, dn1, g1),
                (a2_r, v2_r, c2a_r, c2b_r, m2, dn2, g2)):
            av = a_r[...]
            cv = ca_r[...] + cb_r[...]
            mask = cv > 0.0
            am = jnp.where(mask, av, -jnp.inf)
            bm = jnp.max(am, axis=0, keepdims=True)
            m_old = m[...]
            m_new = jnp.maximum(m_old, bm)
            r = jnp.exp(jnp.where(m_new == -jnp.inf, 0.0, m_old - m_new))
            eb = jnp.where(mask, cv * jnp.exp(av - m_new), 0.0)
            dn[...] = dn[...] * r + jnp.sum(eb, axis=0, keepdims=True)
            g[...] = g[...] * r + lax.dot_general(
                v_r[...], eb, (((0,), (0,)), ((), ())),
                preferred_element_type=jnp.float32, precision=PREC_F32)
            m[...] = m_new

        @pl.when(j == nblk - 1)
        def _():
            agg1 = jnp.transpose(g1[...] / jnp.maximum(dn1[...], 1e-16), (1, 0))
            agg2 = jnp.transpose(g2[...] / jnp.maximum(dn2[...], 1e-16), (1, 0))
            xgv = xg_r[...]
            hg = (agg1 + jnp.dot(xgv, ws1_r[...], preferred_element_type=jnp.float32, precision=PREC) + bs1_r[...]
                  + agg2 + jnp.dot(xgv, ws2_r[...], preferred_element_type=jnp.float32, precision=PREC) + bs2_r[...])
            y = jnp.dot(hg, wlin_r[...], preferred_element_type=jnp.float32, precision=PREC) + blin_r[...]
            if head is not None:
                y = jnp.tanh(jnp.dot(y, w2_r[...], preferred_element_type=jnp.float32, precision=PREC)
                             + b2_r[...])
            o_r[...] = y

    row = lambda shp: pl.BlockSpec(shp, lambda i: (i, 0))
    full2 = lambda shp: pl.BlockSpec(shp, lambda i: (0, 0))
    full1 = lambda shp: pl.BlockSpec(shp, lambda i: (0,))
    in_specs = [
        row((B, NG)), row((B, HID)), row((B, NG)), row((B, NG)),
        row((B, NG)), row((B, HID)), row((B, NG)), row((B, NG)),
        full2((NG, din)),
        full2((din, HID)), full1((HID,)), full2((din, HID)), full1((HID,)),
        full2((HID, HID)), full1((HID,)),
    ]
    args = [a1, v1, c1a, c1b, a2, v2, c2a, c2b, xg,
            tc1['Ws'], tc1['bs'], tc2['Ws'], tc2['bs'], lin['W'], lin['b']]
    if head is not None:
        in_specs += [full2((HID, OUT)), full1((OUT,))]
        args += [head['W'], head['b']]
    return pl.pallas_call(
        body,
        grid=(nblk,),
        in_specs=in_specs,
        out_specs=full2((NG, dout)),
        out_shape=jax.ShapeDtypeStruct((NG, dout), jnp.float32),
        scratch_shapes=[pltpu.VMEM((1, NG), jnp.float32),
                        pltpu.VMEM((1, NG), jnp.float32),
                        pltpu.VMEM((HID, NG), jnp.float32)] * 2,
        interpret=interpret,
    )(*args)


def kernel(x_phylonodes_up, x_phylonodes_down, x_godnode,
           edge_index_up_up, edge_index_down_down, edge_index_down_up,
           edge_index_up_down, edge_index_down_god, edge_index_up_god, params):
    z1 = jnp.zeros((NB_DEG // NS,), jnp.float32)
    z1p = jnp.zeros((NB_PAIR // NS,), jnp.float32)
    z128 = jnp.zeros((NP // NS, 128), jnp.float32)

    bc_deg = _build_bincount(NB_DEG, use_pair=False)
    bc_pair = _build_bincount(NB_PAIR, use_pair=True)
    seg1 = _build_segsum(feature_split=False)
    seg2 = _build_segsum(feature_split=True)

    eis = {
        'uu': edge_index_up_up, 'dd': edge_index_down_down,
        'du': edge_index_down_up, 'ud': edge_index_up_down,
        'dg': edge_index_down_god, 'ug': edge_index_up_god,
    }
    src = {k: v[0] for k, v in eis.items()}
    dst = {k: v[1] for k, v in eis.items()}

    deg = {}
    for t in ('uu', 'du', 'dd', 'ud'):
        cnt = bc_deg(src[t], dst[t], z1).reshape(NC, NB_DEG)
        deg[t] = (cnt[0, :N].reshape(N, 1), cnt[1, :N].reshape(N, 1))
    cmat = {}
    for t in ('dg', 'ug'):
        cnt = bc_pair(src[t], dst[t], z1p).reshape(NC, NB_PAIR)
        cc = cnt[:, :N * NG].reshape(NC, N, NG)
        cmat[t] = (cc[0], cc[1])

    x_up, x_down, x_god = x_phylonodes_up, x_phylonodes_down, x_godnode
    for li, lp in enumerate(params['layers']):
        din = x_up.shape[1]
        if din == DF:
            seg = seg1
            xu = (x_up, x_up)
            xd = (x_down, x_down)
        else:
            seg = seg2
            xu = (x_up[:, :128], x_up[:, 128:])
            xd = (x_down[:, :128], x_down[:, 128:])
        h = {}
        for t, xs in (('uu', xu), ('du', xd), ('dd', xd), ('ud', xu)):
            hh = seg(xs[0], xs[1], src[t], dst[t], z128)
            h[t] = (hh[0, :N], hh[1, :N])
        final = li == len(params['layers']) - 1
        head_up = params['lins2']['up'] if final else None
        head_down = params['lins2']['down'] if final else None
        head_god = params['lins2']['god'] if final else None
        new_up = _mf_layer(h['uu'][0], h['uu'][1], h['du'][0], h['du'][1], x_up,
                           deg['uu'][0], deg['uu'][1], deg['du'][0], deg['du'][1],
                           lp['mf_uu'], lp['mf_du'], lp['lin_up'], head_up)
        new_down = _mf_layer(h['dd'][0], h['dd'][1], h['ud'][0], h['ud'][1], x_down,
                             deg['dd'][0], deg['dd'][1], deg['ud'][0], deg['ud'][1],
                             lp['mf_dd'], lp['mf_ud'], lp['lin_down'], head_down)
        a1, v1 = _qkv(x_down, x_god, lp['tc_dg'])
        a2, v2 = _qkv(x_up, x_god, lp['tc_ug'])
        new_god = _god_layer(a1, v1, cmat['dg'][0], cmat['dg'][1],
                             a2, v2, cmat['ug'][0], cmat['ug'][1], x_god,
                             lp['tc_dg'], lp['tc_ug'], lp['lin_god'], head_god)
        x_up, x_down, x_god = new_up, new_down, new_god
    return (x_up, x_down, x_god)
